# baseline (speedup 1.0000x reference)
import functools
import jax
import jax.numpy as jnp
from jax import lax
from jax.experimental import pallas as pl
from jax.experimental.pallas import tpu as pltpu
from jax.experimental.pallas import tpu_sc as plsc

N_NODES = 10000
N_EDGES = 320000
IN_DIM = 128
HID_DIM = 128
Z_DIM = 64

NC = 2
NS = 16
LANES = 16
NW = NC * NS

E_BLK = 128
N_BLK = 80
E_PER_W = N_BLK * E_BLK
E_PAD = NW * E_PER_W
AGG_ROWS = 10240
ROWS_PER_TILE = AGG_ROWS // NS
ZCHUNK = 80


def _mm_body(x_ref, w_ref, b_ref, o_ref):
    acc = lax.dot_general(x_ref[...], w_ref[...],
                          (((1,), (1,)), ((), ())),
                          preferred_element_type=jnp.float32)
    o_ref[...] = acc + b_ref[...][None, :]


def _shared_linear(x, w, b):
    blk = 1000
    grid = N_NODES // blk
    return pl.pallas_call(
        _mm_body,
        grid=(grid,),
        in_specs=[
            pl.BlockSpec((blk, IN_DIM), lambda i: (i, 0)),
            pl.BlockSpec((HID_DIM, IN_DIM), lambda i: (0, 0)),
            pl.BlockSpec((HID_DIM,), lambda i: (0,)),
        ],
        out_specs=pl.BlockSpec((blk, HID_DIM), lambda i: (i, 0)),
        out_shape=jax.ShapeDtypeStruct((N_NODES, HID_DIM), jnp.float32),
    )(x, w, b)


def _spmm_body(h_hbm, src_hbm, dst_hbm, adj_hbm, out_hbm,
               src0_v, dst0_v, adj0_v, rows0_v,
               src1_v, dst1_v, adj1_v, rows1_v,
               agg_sh, gsem0, gsem1, ssem0, ssem1):
    cid = lax.axis_index("c")
    sid = lax.axis_index("s")
    wid = sid * NC + cid

    def zrow(i, _):
        for j in range(HID_DIM // LANES):
            rows0_v[i, pl.ds(j * LANES, LANES)] = jnp.zeros((LANES,), jnp.float32)
        return 0
    lax.fori_loop(0, E_BLK, zrow, 0)
    for k in range(ROWS_PER_TILE // E_BLK):
        pltpu.sync_copy(rows0_v, agg_sh.at[pl.ds(sid * ROWS_PER_TILE + k * E_BLK, E_BLK)])
    plsc.subcore_barrier()

    ebase = wid * E_PER_W
    srcs = (src0_v, src1_v)
    dsts = (dst0_v, dst1_v)
    adjs = (adj0_v, adj1_v)
    rows = (rows0_v, rows1_v)
    gsems = (gsem0, gsem1)
    ssems = (ssem0, ssem1)

    def stage(b, s, first):
        if not first:
            pltpu.make_async_copy(rows[s], agg_sh.at[dsts[s]], ssems[s]).wait()
        base = ebase + b * E_BLK
        pltpu.sync_copy(src_hbm.at[pl.ds(base, E_BLK)], srcs[s])
        pltpu.sync_copy(dst_hbm.at[pl.ds(base, E_BLK)], dsts[s])
        pltpu.sync_copy(adj_hbm.at[pl.ds(base, E_BLK)], adjs[s])
        pltpu.async_copy(h_hbm.at[srcs[s]], rows[s], gsems[s])

    def compute(s):
        pltpu.make_async_copy(h_hbm.at[srcs[s]], rows[s], gsems[s]).wait()
        rv = rows[s]
        av_ref = adjs[s]

        def group(g, _):
            av = av_ref[pl.ds(g * LANES, LANES)]
            for i in range(LANES):
                e = g * LANES + i
                scale = jnp.broadcast_to(av[i], (LANES,))
                for j in range(HID_DIM // LANES):
                    seg = rv[e, pl.ds(j * LANES, LANES)]
                    rv[e, pl.ds(j * LANES, LANES)] = seg * scale
            return 0
        lax.fori_loop(0, E_BLK // LANES, group, 0)
        pltpu.async_copy(rv, agg_sh.at[dsts[s]], ssems[s], add=True)

    stage(0, 0, True)
    stage(1, 1, True)

    def pair(p, _):
        b0 = 2 * p
        compute(0)

        @pl.when(p < N_BLK // 2 - 1)
        def _():
            stage(b0 + 2, 0, False)
        compute(1)

        @pl.when(p < N_BLK // 2 - 1)
        def _():
            stage(b0 + 3, 1, False)
        return 0

    lax.fori_loop(0, N_BLK // 2, pair, 0)
    pltpu.make_async_copy(rows[0], agg_sh.at[dsts[0]], ssems[0]).wait()
    pltpu.make_async_copy(rows[1], agg_sh.at[dsts[1]], ssems[1]).wait()
    plsc.subcore_barrier()

    rbase = sid * ROWS_PER_TILE
    pltpu.sync_copy(agg_sh.at[pl.ds(rbase, ROWS_PER_TILE)],
                    out_hbm.at[cid].at[pl.ds(rbase, ROWS_PER_TILE)])


def _spmm(h, src, dst, adj):
    mesh = plsc.VectorSubcoreMesh(core_axis_name="c", subcore_axis_name="s")
    k = functools.partial(
        pl.kernel,
        out_type=jax.ShapeDtypeStruct((NC, AGG_ROWS, HID_DIM), jnp.float32),
        mesh=mesh,
        scratch_types=[
            pltpu.VMEM((E_BLK,), jnp.int32),
            pltpu.VMEM((E_BLK,), jnp.int32),
            pltpu.VMEM((E_BLK,), jnp.float32),
            pltpu.VMEM((E_BLK, HID_DIM), jnp.float32),
            pltpu.VMEM((E_BLK,), jnp.int32),
            pltpu.VMEM((E_BLK,), jnp.int32),
            pltpu.VMEM((E_BLK,), jnp.float32),
            pltpu.VMEM((E_BLK, HID_DIM), jnp.float32),
            pltpu.VMEM_SHARED((AGG_ROWS, HID_DIM), jnp.float32),
            pltpu.SemaphoreType.DMA,
            pltpu.SemaphoreType.DMA,
            pltpu.SemaphoreType.DMA,
            pltpu.SemaphoreType.DMA,
        ],
    )(_spmm_body)
    pad = E_PAD - N_EDGES
    srcp = jnp.concatenate([src, jnp.zeros((pad,), jnp.int32)])
    dstp = jnp.concatenate([dst, jnp.zeros((pad,), jnp.int32)])
    adjp = jnp.concatenate([adj, jnp.zeros((pad,), jnp.float32)])
    return k(h, srcp, dstp, adjp)


def _heads_body(p0_ref, p1_ref, wm1_ref, bm1_ref, wm2_ref, bm2_ref,
                ws1_ref, bs1_ref, ws2_ref, bs2_ref, mean_ref, std_ref):
    hidden = jnp.maximum(p0_ref[...] + p1_ref[...], 0.0)
    dn = (((1,), (1,)), ((), ()))
    a = jnp.maximum(
        lax.dot_general(hidden, wm1_ref[...], dn, preferred_element_type=jnp.float32)
        + bm1_ref[...][None, :], 0.0)
    mean_ref[...] = (lax.dot_general(a, wm2_ref[...], dn, preferred_element_type=jnp.float32)
                     + bm2_ref[...][None, :])
    s = jnp.maximum(
        lax.dot_general(hidden, ws1_ref[...], dn, preferred_element_type=jnp.float32)
        + bs1_ref[...][None, :], 0.0)
    pre = (lax.dot_general(s, ws2_ref[...], dn, preferred_element_type=jnp.float32)
           + bs2_ref[...][None, :])
    std_ref[...] = jnp.maximum(pre, 0.0) + jnp.log1p(jnp.exp(-jnp.abs(pre)))


def _heads(partials, wm1, bm1, wm2, bm2, ws1, bs1, ws2, bs2):
    blk = 1024
    grid = AGG_ROWS // blk
    wspec = lambda shape: pl.BlockSpec(shape, lambda i: tuple(0 for _ in shape))
    mean, std = pl.pallas_call(
        _heads_body,
        grid=(grid,),
        in_specs=[
            pl.BlockSpec((blk, HID_DIM), lambda i: (i, 0)),
            pl.BlockSpec((blk, HID_DIM), lambda i: (i, 0)),
            wspec((Z_DIM, HID_DIM)), wspec((Z_DIM,)),
            wspec((Z_DIM, Z_DIM)), wspec((Z_DIM,)),
            wspec((Z_DIM, HID_DIM)), wspec((Z_DIM,)),
            wspec((Z_DIM, Z_DIM)), wspec((Z_DIM,)),
        ],
        out_specs=[
            pl.BlockSpec((blk, Z_DIM), lambda i: (i, 0)),
            pl.BlockSpec((blk, Z_DIM), lambda i: (i, 0)),
        ],
        out_shape=[
            jax.ShapeDtypeStruct((AGG_ROWS, Z_DIM), jnp.float32),
            jax.ShapeDtypeStruct((AGG_ROWS, Z_DIM), jnp.float32),
        ],
    )(partials[0], partials[1],
      wm1, bm1, wm2, bm2, ws1, bs1, ws2, bs2)
    return mean, std


def kernel(x, edge_index, adj_values, W_shared, b_shared,
           W_m1, b_m1, W_m2, b_m2, W_s1, b_s1, W_s2, b_s2):
    ei = edge_index.astype(jnp.int32)
    dst = ei[0]
    src = ei[1]
    h = _shared_linear(x, W_shared, b_shared)
    partials = _spmm(h, src, dst, adj_values)
    mean, std = _heads(partials, W_m1, b_m1, W_m2, b_m2, W_s1, b_s1, W_s2, b_s2)
    mean = mean[:N_NODES]
    std = std[:N_NODES]
    return (mean, mean, std)


# speedup vs baseline: 1.5155x; 1.5155x over previous
import functools
import jax
import jax.numpy as jnp
from jax import lax
from jax.experimental import pallas as pl
from jax.experimental.pallas import tpu as pltpu
from jax.experimental.pallas import tpu_sc as plsc

N_NODES = 10000
N_EDGES = 320000
IN_DIM = 128
HID_DIM = 128
Z_DIM = 64

NC = 2
NS = 16
LANES = 16
NW = NC * NS

E_PER_W = N_EDGES // NW
E_BLK = 80
N_BLK = E_PER_W // E_BLK
AGG_ROWS = 10240
ROWS_PER_TILE = AGG_ROWS // NS
ZCHUNK = 80


def _mm_body(x_ref, w_ref, b_ref, o_ref):
    acc = lax.dot_general(x_ref[...], w_ref[...],
                          (((1,), (1,)), ((), ())),
                          preferred_element_type=jnp.float32)
    o_ref[...] = acc + b_ref[...][None, :]


def _shared_linear(x, w, b):
    blk = 1000
    grid = N_NODES // blk
    return pl.pallas_call(
        _mm_body,
        grid=(grid,),
        in_specs=[
            pl.BlockSpec((blk, IN_DIM), lambda i: (i, 0)),
            pl.BlockSpec((HID_DIM, IN_DIM), lambda i: (0, 0)),
            pl.BlockSpec((HID_DIM,), lambda i: (0,)),
        ],
        out_specs=pl.BlockSpec((blk, HID_DIM), lambda i: (i, 0)),
        out_shape=jax.ShapeDtypeStruct((N_NODES, HID_DIM), jnp.float32),
    )(x, w, b)


def _spmm_body(h_hbm, src_hbm, dst_hbm, adj_hbm, out_hbm,
               src0_v, dst0_v, adj0_v, rows0_v,
               src1_v, dst1_v, adj1_v, rows1_v,
               zbuf_v, agg_sh, gsem0, gsem1, ssem0, ssem1):
    cid = lax.axis_index("c")
    sid = lax.axis_index("s")
    wid = sid * NC + cid

    def zrow(i, _):
        for j in range(HID_DIM // LANES):
            zbuf_v[i, pl.ds(j * LANES, LANES)] = jnp.zeros((LANES,), jnp.float32)
        return 0
    lax.fori_loop(0, ZCHUNK, zrow, 0)
    for k in range(ROWS_PER_TILE // ZCHUNK):
        pltpu.sync_copy(zbuf_v, agg_sh.at[pl.ds(sid * ROWS_PER_TILE + k * ZCHUNK, ZCHUNK)])
    plsc.subcore_barrier()

    ebase = wid * E_PER_W
    srcs = (src0_v, src1_v)
    dsts = (dst0_v, dst1_v)
    adjs = (adj0_v, adj1_v)
    rows = (rows0_v, rows1_v)
    gsems = (gsem0, gsem1)
    ssems = (ssem0, ssem1)

    def stage(b, s, first):
        if not first:
            pltpu.make_async_copy(rows[s], agg_sh.at[dsts[s]], ssems[s]).wait()
        base = ebase + b * E_BLK
        pltpu.sync_copy(src_hbm.at[pl.ds(base, E_BLK)], srcs[s])
        pltpu.sync_copy(dst_hbm.at[pl.ds(base, E_BLK)], dsts[s])
        pltpu.sync_copy(adj_hbm.at[pl.ds(base, E_BLK)], adjs[s])
        pltpu.async_copy(h_hbm.at[srcs[s]], rows[s], gsems[s])

    def compute(s):
        pltpu.make_async_copy(h_hbm.at[srcs[s]], rows[s], gsems[s]).wait()
        rv = rows[s]
        av_ref = adjs[s]

        def group(g, _):
            av = av_ref[pl.ds(g * LANES, LANES)]
            for i in range(LANES):
                e = g * LANES + i
                scale = jnp.broadcast_to(av[i], (LANES,))
                for j in range(HID_DIM // LANES):
                    seg = rv[e, pl.ds(j * LANES, LANES)]
                    rv[e, pl.ds(j * LANES, LANES)] = seg * scale
            return 0
        lax.fori_loop(0, E_BLK // LANES, group, 0)
        pltpu.async_copy(rv, agg_sh.at[dsts[s]], ssems[s], add=True)

    stage(0, 0, True)
    stage(1, 1, True)

    def pair(p, _):
        b0 = 2 * p
        compute(0)
        stage(b0 + 2, 0, False)
        compute(1)

        @pl.when(p < (N_BLK - 1) // 2 - 1)
        def _():
            stage(b0 + 3, 1, False)
        return 0

    lax.fori_loop(0, (N_BLK - 1) // 2, pair, 0)
    compute(0)
    pltpu.make_async_copy(rows[0], agg_sh.at[dsts[0]], ssems[0]).wait()
    pltpu.make_async_copy(rows[1], agg_sh.at[dsts[1]], ssems[1]).wait()
    plsc.subcore_barrier()

    rbase = sid * ROWS_PER_TILE
    pltpu.sync_copy(agg_sh.at[pl.ds(rbase, ROWS_PER_TILE)],
                    out_hbm.at[cid].at[pl.ds(rbase, ROWS_PER_TILE)])


def _spmm(h, src, dst, adj):
    mesh = plsc.VectorSubcoreMesh(core_axis_name="c", subcore_axis_name="s")
    k = functools.partial(
        pl.kernel,
        out_type=jax.ShapeDtypeStruct((NC, AGG_ROWS, HID_DIM), jnp.float32),
        mesh=mesh,
        scratch_types=[
            pltpu.VMEM((E_BLK,), jnp.int32),
            pltpu.VMEM((E_BLK,), jnp.int32),
            pltpu.VMEM((E_BLK,), jnp.float32),
            pltpu.VMEM((E_BLK, HID_DIM), jnp.float32),
            pltpu.VMEM((E_BLK,), jnp.int32),
            pltpu.VMEM((E_BLK,), jnp.int32),
            pltpu.VMEM((E_BLK,), jnp.float32),
            pltpu.VMEM((E_BLK, HID_DIM), jnp.float32),
            pltpu.VMEM((ZCHUNK, HID_DIM), jnp.float32),
            pltpu.VMEM_SHARED((AGG_ROWS, HID_DIM), jnp.float32),
            pltpu.SemaphoreType.DMA,
            pltpu.SemaphoreType.DMA,
            pltpu.SemaphoreType.DMA,
            pltpu.SemaphoreType.DMA,
        ],
    )(_spmm_body)
    return k(h, src, dst, adj)


def _heads_body(p0_ref, p1_ref, wm1_ref, bm1_ref, wm2_ref, bm2_ref,
                ws1_ref, bs1_ref, ws2_ref, bs2_ref, mean_ref, std_ref):
    hidden = jnp.maximum(p0_ref[...] + p1_ref[...], 0.0)
    dn = (((1,), (1,)), ((), ()))
    a = jnp.maximum(
        lax.dot_general(hidden, wm1_ref[...], dn, preferred_element_type=jnp.float32)
        + bm1_ref[...][None, :], 0.0)
    mean_ref[...] = (lax.dot_general(a, wm2_ref[...], dn, preferred_element_type=jnp.float32)
                     + bm2_ref[...][None, :])
    s = jnp.maximum(
        lax.dot_general(hidden, ws1_ref[...], dn, preferred_element_type=jnp.float32)
        + bs1_ref[...][None, :], 0.0)
    pre = (lax.dot_general(s, ws2_ref[...], dn, preferred_element_type=jnp.float32)
           + bs2_ref[...][None, :])
    std_ref[...] = jnp.maximum(pre, 0.0) + jnp.log1p(jnp.exp(-jnp.abs(pre)))


def _heads(partials, wm1, bm1, wm2, bm2, ws1, bs1, ws2, bs2):
    blk = 1024
    grid = AGG_ROWS // blk
    wspec = lambda shape: pl.BlockSpec(shape, lambda i: tuple(0 for _ in shape))
    mean, std = pl.pallas_call(
        _heads_body,
        grid=(grid,),
        in_specs=[
            pl.BlockSpec((blk, HID_DIM), lambda i: (i, 0)),
            pl.BlockSpec((blk, HID_DIM), lambda i: (i, 0)),
            wspec((Z_DIM, HID_DIM)), wspec((Z_DIM,)),
            wspec((Z_DIM, Z_DIM)), wspec((Z_DIM,)),
            wspec((Z_DIM, HID_DIM)), wspec((Z_DIM,)),
            wspec((Z_DIM, Z_DIM)), wspec((Z_DIM,)),
        ],
        out_specs=[
            pl.BlockSpec((blk, Z_DIM), lambda i: (i, 0)),
            pl.BlockSpec((blk, Z_DIM), lambda i: (i, 0)),
        ],
        out_shape=[
            jax.ShapeDtypeStruct((AGG_ROWS, Z_DIM), jnp.float32),
            jax.ShapeDtypeStruct((AGG_ROWS, Z_DIM), jnp.float32),
        ],
    )(partials[0], partials[1],
      wm1, bm1, wm2, bm2, ws1, bs1, ws2, bs2)
    return mean, std


def kernel(x, edge_index, adj_values, W_shared, b_shared,
           W_m1, b_m1, W_m2, b_m2, W_s1, b_s1, W_s2, b_s2):
    ei = edge_index.astype(jnp.int32)
    dst = ei[0]
    src = ei[1]
    h = _shared_linear(x, W_shared, b_shared)
    partials = _spmm(h, src, dst, adj_values)
    mean, std = _heads(partials, W_m1, b_m1, W_m2, b_m2, W_s1, b_s1, W_s2, b_s2)
    mean = mean[:N_NODES]
    std = std[:N_NODES]
    return (mean, mean, std)
